# Initial kernel scaffold; baseline (speedup 1.0000x reference)
import jax
import jax.numpy as jnp
from jax.experimental import pallas as pl


def kernel(pose, params, codebook):
    raise NotImplementedError("write your pallas kernel here")


# speedup vs baseline: 2.4371x; 2.4371x over previous
import functools

import jax
import jax.numpy as jnp
from jax.experimental import pallas as pl

NUM_JOINTS = 24
INPUT_DIM = 9
HID = 512
TOK_INTER = 64
N_MIX = 4
TOKEN_NUM = 34
TOKEN_CLASS = 2048
TOKEN_DIM = 512
CB_TILE = 512

BBLK = 32

_BF16 = jnp.bfloat16
_F32 = jnp.float32


def _mm(a, b16):
    return jax.lax.dot_general(
        a.astype(_BF16) if a.dtype != _BF16 else a, b16,
        (((1,), (0,)), ((), ())), preferred_element_type=_F32)


def _ln(x, g, b, eps=1e-5):
    m = jnp.mean(x, axis=-1, keepdims=True)
    v = jnp.mean((x - m) ** 2, axis=-1, keepdims=True)
    return (x - m) / jnp.sqrt(v + eps) * g + b


def _gelu(x):
    return 0.5 * x * (1.0 + jax.lax.erf(x * jnp.float32(0.7071067811865476)))


def _token_mix(y2d, bblk, t_in, w1, b1, w2, b2):
    y3 = y2d.reshape(bblk, t_in, HID)
    yt = jnp.swapaxes(y3, 1, 2).reshape(bblk * HID, t_in)
    h = _gelu(_mm(yt, w1) + b1)
    z = _mm(h, w2) + b2
    t_out = z.shape[-1]
    z3 = z.reshape(bblk, HID, t_out)
    return jnp.swapaxes(z3, 1, 2).reshape(bblk * t_out, HID)


def _mixer(x2d, bblk, t, ln1_g, ln1_b, tw1, tb1, tw2, tb2,
           ln2_g, ln2_b, cw1, cb1, cw2, cb2):
    y = _ln(x2d, ln1_g, ln1_b)
    y = _token_mix(y, bblk, t, tw1, tb1, tw2, tb2)
    x1 = x2d + y
    z = _ln(x1, ln2_g, ln2_b)
    z = _mm(z, cw1) + cb1
    z = _gelu(z)
    z = _mm(z, cw2) + cb2
    return x1 + z


def _fused_kernel(
    pose_ref,
    w_start_ref, b_start_ref,
    enc_ln1g_ref, enc_ln1b_ref, enc_tw1_ref, enc_tb1_ref, enc_tw2_ref,
    enc_tb2_ref, enc_ln2g_ref, enc_ln2b_ref, enc_cw1_ref, enc_cb1_ref,
    enc_cw2_ref, enc_cb2_ref,
    enc_lnfg_ref, enc_lnfb_ref,
    w_tok_ref, b_tok_ref, w_feat_ref, b_feat_ref,
    w_dtok_ref, b_dtok_ref, w_dstart_ref, b_dstart_ref,
    dec_ln1g_ref, dec_ln1b_ref, dec_tw1_ref, dec_tb1_ref, dec_tw2_ref,
    dec_tb2_ref, dec_ln2g_ref, dec_ln2b_ref, dec_cw1_ref, dec_cb1_ref,
    dec_cw2_ref, dec_cb2_ref,
    dec_lnfg_ref, dec_lnfb_ref,
    w_rec_ref, b_rec_ref,
    cbt_f32_ref, cbt16_ref, cb16_ref,
    rec_ref, idx_ref, elat_ref,
):
    step = pl.program_id(0)
    bblk = pose_ref.shape[0]

    pose2d = pose_ref[...].reshape(bblk * NUM_JOINTS, INPUT_DIM)
    f = _mm(pose2d, w_start_ref[...]) + b_start_ref[...]

    for i in range(N_MIX):
        f = _mixer(
            f, bblk, NUM_JOINTS,
            enc_ln1g_ref[i], enc_ln1b_ref[i],
            enc_tw1_ref[i], enc_tb1_ref[i], enc_tw2_ref[i], enc_tb2_ref[i],
            enc_ln2g_ref[i], enc_ln2b_ref[i],
            enc_cw1_ref[i], enc_cb1_ref[i], enc_cw2_ref[i], enc_cb2_ref[i])

    f = _ln(f, enc_lnfg_ref[...], enc_lnfb_ref[...])
    f3 = f.reshape(bblk, NUM_JOINTS, HID)
    ft = jnp.swapaxes(f3, 1, 2).reshape(bblk * HID, NUM_JOINTS)
    g = _mm(ft, w_tok_ref[...]) + b_tok_ref[...]
    g3 = g.reshape(bblk, HID, TOKEN_NUM)
    g = jnp.swapaxes(g3, 1, 2).reshape(bblk * TOKEN_NUM, HID)
    enc = _mm(g, w_feat_ref[...]) + b_feat_ref[...]
    rows = bblk * TOKEN_NUM

    enc16 = enc.astype(_BF16)
    enc_sq = jnp.sum(enc * enc, axis=1, keepdims=True)
    best = jnp.full((rows, 1), jnp.inf, _F32)
    bidx = jnp.zeros((rows, 1), jnp.int32)
    n_tiles = TOKEN_CLASS // CB_TILE
    for t in range(n_tiles):
        cbt_f32 = cbt_f32_ref[:, t * CB_TILE:(t + 1) * CB_TILE]
        cb_sq = jnp.sum(cbt_f32 * cbt_f32, axis=0, keepdims=True)
        sc = jax.lax.dot_general(
            enc16, cbt16_ref[:, t * CB_TILE:(t + 1) * CB_TILE],
            (((1,), (0,)), ((), ())), preferred_element_type=_F32)
        d = enc_sq + cb_sq - 2.0 * sc
        m = jnp.min(d, axis=1, keepdims=True)
        col = jax.lax.broadcasted_iota(jnp.int32, (rows, CB_TILE), 1)
        a = jnp.min(jnp.where(d == m, col + t * CB_TILE, TOKEN_CLASS),
                    axis=1, keepdims=True)
        upd = m < best
        best = jnp.where(upd, m, best)
        bidx = jnp.where(upd, a, bidx)

    idx_ref[...] = bidx

    quant = jnp.zeros((rows, TOKEN_DIM), _F32)
    for t in range(n_tiles):
        col = jax.lax.broadcasted_iota(jnp.int32, (rows, CB_TILE), 1)
        oh = (bidx == col + t * CB_TILE).astype(_BF16)
        quant = quant + jax.lax.dot_general(
            oh, cb16_ref[t * CB_TILE:(t + 1) * CB_TILE, :],
            (((1,), (0,)), ((), ())), preferred_element_type=_F32)

    psum = jnp.sum((quant - enc) ** 2)

    @pl.when(step == 0)
    def _init():
        elat_ref[...] = jnp.zeros_like(elat_ref)

    elat_ref[0, 0, :] = elat_ref[0, 0, :] + psum

    q3 = quant.reshape(bblk, TOKEN_NUM, TOKEN_DIM)
    qt = jnp.swapaxes(q3, 1, 2).reshape(bblk * TOKEN_DIM, TOKEN_NUM)
    p = _mm(qt, w_dtok_ref[...]) + b_dtok_ref[...]
    p3 = p.reshape(bblk, TOKEN_DIM, NUM_JOINTS)
    p = jnp.swapaxes(p3, 1, 2).reshape(bblk * NUM_JOINTS, TOKEN_DIM)
    df = _mm(p, w_dstart_ref[...]) + b_dstart_ref[...]

    for i in range(N_MIX):
        df = _mixer(
            df, bblk, NUM_JOINTS,
            dec_ln1g_ref[i], dec_ln1b_ref[i],
            dec_tw1_ref[i], dec_tb1_ref[i], dec_tw2_ref[i], dec_tb2_ref[i],
            dec_ln2g_ref[i], dec_ln2b_ref[i],
            dec_cw1_ref[i], dec_cb1_ref[i], dec_cw2_ref[i], dec_cb2_ref[i])

    df = _ln(df, dec_lnfg_ref[...], dec_lnfb_ref[...])
    rec = _mm(df, w_rec_ref[...]) + b_rec_ref[...]
    rec_ref[...] = rec.reshape(bblk, NUM_JOINTS, INPUT_DIM)


def _row(x):
    return x.reshape(1, -1)


def kernel(pose, params, codebook):
    bs = pose.shape[0]
    n_steps = bs // BBLK
    p = params
    bf = lambda x: x.astype(_BF16)

    enc, dec = p['enc'], p['dec']
    inputs = [
        pose,
        bf(p['W_start']), _row(p['b_start']),
        enc['ln1_g'], enc['ln1_b'], bf(enc['tW1']), enc['tb1'],
        bf(enc['tW2']), enc['tb2'], enc['ln2_g'], enc['ln2_b'],
        bf(enc['cW1']), enc['cb1'], bf(enc['cW2']), enc['cb2'],
        _row(p['enc_lnf_g']), _row(p['enc_lnf_b']),
        bf(p['W_tok']), _row(p['b_tok']), bf(p['W_feat']), _row(p['b_feat']),
        bf(p['W_dtok']), _row(p['b_dtok']),
        bf(p['W_dstart']), _row(p['b_dstart']),
        dec['ln1_g'], dec['ln1_b'], bf(dec['tW1']), dec['tb1'],
        bf(dec['tW2']), dec['tb2'], dec['ln2_g'], dec['ln2_b'],
        bf(dec['cW1']), dec['cb1'], bf(dec['cW2']), dec['cb2'],
        _row(p['dec_lnf_g']), _row(p['dec_lnf_b']),
        bf(p['W_rec']), _row(p['b_rec']),
        codebook.T, bf(codebook.T), bf(codebook),
    ]

    def const_spec(x):
        nd = x.ndim
        return pl.BlockSpec(x.shape, lambda i, _n=nd: (0,) * _n)

    in_specs = [pl.BlockSpec((BBLK, NUM_JOINTS, INPUT_DIM),
                             lambda i: (i, 0, 0))]
    in_specs += [const_spec(x) for x in inputs[1:]]

    out_shapes = (
        jax.ShapeDtypeStruct((bs, NUM_JOINTS, INPUT_DIM), _F32),
        jax.ShapeDtypeStruct((bs * TOKEN_NUM, 1), jnp.int32),
        jax.ShapeDtypeStruct((1, 1, 128), _F32),
    )
    out_specs = (
        pl.BlockSpec((BBLK, NUM_JOINTS, INPUT_DIM), lambda i: (i, 0, 0)),
        pl.BlockSpec((BBLK * TOKEN_NUM, 1), lambda i: (i, 0)),
        pl.BlockSpec((1, 1, 128), lambda i: (0, 0, 0)),
    )

    rec, idx2d, elat = pl.pallas_call(
        _fused_kernel,
        grid=(n_steps,),
        in_specs=in_specs,
        out_specs=out_specs,
        out_shape=out_shapes,
    )(*inputs)

    idx = idx2d.reshape(bs * TOKEN_NUM)
    e_latent_loss = elat[0, 0, 0] / jnp.float32(bs * TOKEN_NUM * TOKEN_DIM)
    return rec, idx, e_latent_loss
